# Initial kernel scaffold; baseline (speedup 1.0000x reference)
import jax
import jax.numpy as jnp
from jax.experimental import pallas as pl


def kernel(x, edge_index, batch, W1, b1, W2, b2, W3, b3, Wc1, Wc2):
    raise NotImplementedError("write your pallas kernel here")


# speedup vs baseline: 8.7023x; 8.7023x over previous
import functools

import jax
import jax.numpy as jnp
from jax import lax
from jax.experimental import pallas as pl
from jax.experimental.pallas import tpu as pltpu
from jax.experimental.pallas import tpu_sc as plsc

N = 10000
E = 320000
F = 128
H = 32
G = 64
K = 50
NITER = 3

NC = 2
NS = 16
NW = NC * NS
LANES = 16

NPAD = 10240
CH = 128
NCHUNK = 79
EPT = CH * NCHUNK
EPAD = EPT * NW
TRASH = N

RT = 256
NRT = NPAD // RT

PSLOT = G * K
PROWS = PSLOT * 3
PTRASH = PROWS
PSP = PROWS + LANES

@functools.cache
def _sc_kernels():
    mesh = plsc.VectorSubcoreMesh(
        core_axis_name="c", subcore_axis_name="s",
        num_cores=NC, num_subcores=NS)
    params = pltpu.CompilerParams(use_tc_tiling_on_sc=False)
    deg_k = functools.partial(
        pl.kernel,
        out_type=(
            jax.ShapeDtypeStruct((EPAD,), jnp.int32),
            jax.ShapeDtypeStruct((NC, NPAD), jnp.float32),
        ),
        mesh=mesh,
        scratch_types=[
            pltpu.VMEM((1, CH), jnp.int32),
            pltpu.VMEM((1, CH), jnp.int32),
            pltpu.VMEM((1, CH), jnp.float32),
            pltpu.VMEM_SHARED((NPAD,), jnp.float32),
        ],
        compiler_params=params,
    )(_sc_deg_body)
    prop_k = functools.partial(
        pl.kernel,
        out_type=jax.ShapeDtypeStruct((NC, NPAD, H), jnp.float32),
        mesh=mesh,
        scratch_types=[
            pltpu.VMEM((1, CH), jnp.int32),
            pltpu.VMEM((1, CH), jnp.int32),
            pltpu.VMEM((CH, H), jnp.float32),
            pltpu.VMEM_SHARED((NPAD, H), jnp.float32),
            pltpu.SemaphoreType.DMA,
        ],
        compiler_params=params,
    )(_sc_prop_body)
    pool_k = functools.partial(
        pl.kernel,
        out_type=jax.ShapeDtypeStruct((NC, PROWS, H), jnp.float32),
        mesh=mesh,
        scratch_types=[
            pltpu.VMEM((1, 80), jnp.int32),
            pltpu.VMEM((1, 80), jnp.int32),
            pltpu.VMEM((80, H), jnp.float32),
            pltpu.VMEM_SHARED((PSP, H), jnp.float32),
        ],
        compiler_params=params,
    )(_sc_pool_body)
    return deg_k, prop_k, pool_k


def _sc_deg(src, dst, zeros1):
    return _sc_kernels()[0](src, dst, zeros1)


def _sc_prop(hp, src, dstm, zeros2):
    return _sc_kernels()[1](hp, src, dstm, zeros2)


def _sc_pool(x1, x2, x3, bpad, rank, zeros2):
    return _sc_kernels()[2](x1, x2, x3, bpad, rank, zeros2)


def _sc_deg_body(src_hbm, dst_hbm, zeros1_hbm, dstm_hbm, deg_hbm,
                 sbuf, dbuf, ones, deg_sp):
    cid = lax.axis_index("c")
    sid = lax.axis_index("s")
    wid = sid * NC + cid
    pltpu.sync_copy(zeros1_hbm.at[pl.ds(sid * (NPAD // NS), NPAD // NS)],
                    deg_sp.at[pl.ds(sid * (NPAD // NS), NPAD // NS)])
    for g in range(CH // LANES):
        ones[0, pl.ds(g * LANES, LANES)] = jnp.ones((LANES,), jnp.float32)
    plsc.subcore_barrier()

    def body(j, carry):
        base = wid * EPT + j * CH
        pltpu.sync_copy(src_hbm.at[pl.ds(base, CH)], sbuf.at[0])
        pltpu.sync_copy(dst_hbm.at[pl.ds(base, CH)], dbuf.at[0])
        for g in range(CH // LANES):
            s = sbuf[0, pl.ds(g * LANES, LANES)]
            d = dbuf[0, pl.ds(g * LANES, LANES)]
            dbuf[0, pl.ds(g * LANES, LANES)] = jnp.where(
                s == d, jnp.full((LANES,), TRASH, jnp.int32), d)
        pltpu.sync_copy(dbuf.at[0], dstm_hbm.at[pl.ds(base, CH)])
        pltpu.sync_copy(ones.at[0], deg_sp.at[dbuf.at[0]], add=True)
        return carry

    lax.fori_loop(0, NCHUNK, body, 0)
    plsc.subcore_barrier()
    rpt = NPAD // NS
    pltpu.sync_copy(deg_sp.at[pl.ds(sid * rpt, rpt)],
                    deg_hbm.at[cid, pl.ds(sid * rpt, rpt)])


def _sc_prop_body(hp_hbm, src_hbm, dstm_hbm, zeros2_hbm, agg_hbm,
                  sbuf, dbuf, rows, acc_sp, sem):
    cid = lax.axis_index("c")
    sid = lax.axis_index("s")
    wid = sid * NC + cid
    rpt = NPAD // NS
    pltpu.sync_copy(zeros2_hbm.at[pl.ds(sid * rpt, rpt)],
                    acc_sp.at[pl.ds(sid * rpt, rpt)])
    plsc.subcore_barrier()

    def body(j, carry):
        base = wid * EPT + j * CH
        pltpu.sync_copy(src_hbm.at[pl.ds(base, CH)], sbuf.at[0])
        pltpu.sync_copy(dstm_hbm.at[pl.ds(base, CH)], dbuf.at[0])
        pltpu.async_copy(hp_hbm.at[sbuf.at[0]], rows, sem).wait()
        pltpu.sync_copy(rows, acc_sp.at[dbuf.at[0]], add=True)
        return carry

    lax.fori_loop(0, NCHUNK, body, 0)
    plsc.subcore_barrier()
    pltpu.sync_copy(acc_sp.at[pl.ds(sid * rpt, rpt)],
                    agg_hbm.at[cid, pl.ds(sid * rpt, rpt)])


def _sc_pool_body(x1_hbm, x2_hbm, x3_hbm, batch_hbm, rank_hbm, zeros2_hbm,
                  pooled_hbm, bbuf, ibuf, rows, pool_sp):
    cid = lax.axis_index("c")
    sid = lax.axis_index("s")
    wid = sid * NC + cid
    rpt = PROWS // NS
    pltpu.sync_copy(zeros2_hbm.at[pl.ds(sid * rpt, rpt)],
                    pool_sp.at[pl.ds(sid * rpt, rpt)])
    plsc.subcore_barrier()

    npt = NPAD // NW
    for c in range(4):
        base = wid * npt + c * 80
        pltpu.sync_copy(batch_hbm.at[pl.ds(base, 80)], bbuf.at[0])
        pltpu.sync_copy(rank_hbm.at[pl.ds(base, 80)], ibuf.at[0])
        for g in range(5):
            b = bbuf[0, pl.ds(g * LANES, LANES)]
            r = ibuf[0, pl.ds(g * LANES, LANES)]
            node = base + g * LANES + lax.iota(jnp.int32, LANES)
            ok = (r < K) & (node < N)
            slot3 = jnp.where(ok, (r * G + b) * 3,
                              jnp.full((LANES,), PTRASH, jnp.int32))
            ibuf[0, pl.ds(g * LANES, LANES)] = slot3
        for p, tab in ((0, x1_hbm), (1, x2_hbm), (2, x3_hbm)):
            pltpu.sync_copy(tab.at[pl.ds(base, 80)], rows)
            if p:
                for g in range(5):
                    ibuf[0, pl.ds(g * LANES, LANES)] = (
                        ibuf[0, pl.ds(g * LANES, LANES)] + 1)
            pltpu.sync_copy(rows, pool_sp.at[ibuf.at[0]], add=True)
    plsc.subcore_barrier()
    pltpu.sync_copy(pool_sp.at[pl.ds(sid * rpt, rpt)],
                    pooled_hbm.at[cid, pl.ds(sid * rpt, rpt)])


def _tc_pre_body(x_ref, w_ref, d0_ref, d1_ref, h_ref, hp_ref, dis_ref):
    deg = d0_ref[...] + d1_ref[...] + 1.0
    dis = 1.0 / jnp.sqrt(deg)
    h = jnp.dot(x_ref[...], w_ref[...], preferred_element_type=jnp.float32)
    h_ref[...] = h
    hp_ref[...] = h * dis
    dis_ref[...] = dis


def _tc_combine_body(has_next, a0_ref, a1_ref, h_ref, dis_ref, b_ref, wn_ref,
                     *out_refs):
    dis = dis_ref[...]
    agg = a0_ref[...] + a1_ref[...]
    xl = jnp.tanh(agg * dis + h_ref[...] * (dis * dis) + b_ref[...])
    out_refs[0][...] = xl
    if has_next:
        hn = jnp.dot(xl, wn_ref[...], preferred_element_type=jnp.float32)
        out_refs[1][...] = hn
        out_refs[2][...] = hn * dis


def _tc_rank_body(kcol_ref, bcol_ref, k2d_ref, b2d_ref, jlo_ref, jhi_ref,
                  rank_ref):
    i = pl.program_id(0)
    kcol = kcol_ref[...]
    bcol = bcol_ref[...]
    icol = (i * RT + lax.broadcasted_iota(jnp.int32, (RT, 1), 0))

    def body(j, acc):
        krow = k2d_ref[pl.ds(j, 1), :]
        brow = b2d_ref[pl.ds(j, 1), :]
        jrow = j * RT + lax.broadcasted_iota(jnp.int32, (1, RT), 1)
        beq = bcol == brow
        ahead = (krow > kcol) | ((krow == kcol) & (jrow < icol))
        cnt = jnp.sum((beq & ahead).astype(jnp.int32), axis=1, keepdims=True)
        return acc + cnt

    acc0 = jnp.zeros((RT, 1), jnp.int32)
    rank_ref[...] = lax.fori_loop(jlo_ref[i], jhi_ref[i], body, acc0)


def _tc_priors1_body(p0_ref, p1_ref, w1_ref, out_ref, ssum_ref):
    i = pl.program_id(0)
    u = p0_ref[0] + p1_ref[0]
    p = jnp.dot(u, w1_ref[0], preferred_element_type=jnp.float32,
                precision=lax.Precision.HIGHEST)
    out_ref[0] = p

    @pl.when(i == 0)
    def _():
        ssum_ref[...] = jnp.zeros_like(ssum_ref)

    ssum_ref[...] += p


def _squash_flat(s, m, mt):
    n2 = jnp.dot(s * s, m, preferred_element_type=jnp.float32, precision=lax.Precision.HIGHEST)
    f = (n2 / (1.0 + n2)) / jnp.sqrt(n2 + 1e-16)
    return s * jnp.dot(f, mt, preferred_element_type=jnp.float32, precision=lax.Precision.HIGHEST)


def _softmax(l):
    e = jnp.exp(l - jnp.max(l, axis=-1, keepdims=True))
    return e / jnp.sum(e, axis=-1, keepdims=True)


def _tc_squash_body(scale, ssum_ref, m_ref, mt_ref, out_ref):
    out_ref[...] = _squash_flat(ssum_ref[...] * scale, m_ref[...], mt_ref[...])


def _tc_route_step_body(first, p_ref, lprev_ref, out_prev_ref, m_ref, mt_ref,
                        lnew_ref, ssum_ref):
    i = pl.program_id(0)
    p = p_ref[0]
    d = jnp.dot(p * out_prev_ref[...], m_ref[...],
                preferred_element_type=jnp.float32, precision=lax.Precision.HIGHEST)
    l = d if first else lprev_ref[0] + d
    lnew_ref[0] = l
    probs = _softmax(l)
    contrib = p * jnp.dot(probs, mt_ref[...], preferred_element_type=jnp.float32, precision=lax.Precision.HIGHEST)

    @pl.when(i == 0)
    def _():
        ssum_ref[...] = jnp.zeros_like(ssum_ref)

    ssum_ref[...] += contrib


def _tc_caps2_body(ssum_ref, w2_ref, m_ref, mt_ref, m2_ref, mt2_ref, out_ref):
    u2f = _squash_flat(ssum_ref[...], m_ref[...], mt_ref[...])
    u2 = u2f.reshape(G, 16, H)
    pr2 = []
    for i in range(16):
        pr2.append(jnp.dot(u2[:, i, :], w2_ref[i],
                           preferred_element_type=jnp.float32, precision=lax.Precision.HIGHEST))
    p2 = jnp.stack(pr2)
    m2 = m2_ref[...]
    mt2 = mt2_ref[...]
    logits = jnp.zeros((16, G, 10), jnp.float32)
    out = None
    for it in range(NITER):
        probs = _softmax(logits)
        pe = jnp.dot(probs.reshape(16 * G, 10), mt2,
                     preferred_element_type=jnp.float32, precision=lax.Precision.HIGHEST).reshape(16, G, 160)
        s = jnp.sum(pe * p2, axis=0)
        out = _squash_flat(s, m2, mt2)
        if it != NITER - 1:
            d = jnp.dot((p2 * out[None]).reshape(16 * G, 160), m2,
                        preferred_element_type=jnp.float32, precision=lax.Precision.HIGHEST)
            logits = logits + d.reshape(16, G, 10)
    out_ref[...] = jnp.sqrt(jnp.dot(out * out, m2_ref[...],
                                    preferred_element_type=jnp.float32, precision=lax.Precision.HIGHEST))


def _row_spec(cols):
    return pl.BlockSpec((RT, cols), lambda i: (i, 0))


def _full_spec():
    return pl.BlockSpec(memory_space=pltpu.ANY)


def kernel(x, edge_index, batch, W1, b1, W2, b2, W3, b3, Wc1, Wc2):
    f32 = jnp.float32
    src = jnp.concatenate([edge_index[0],
                           jnp.full((EPAD - E,), TRASH, jnp.int32)])
    dst = jnp.concatenate([edge_index[1],
                           jnp.full((EPAD - E,), TRASH, jnp.int32)])
    zeros1 = jnp.zeros((NPAD,), f32)
    zeros2 = jnp.zeros((NPAD, H), f32)
    xpad = jnp.zeros((NPAD, F), f32).at[:N].set(x)
    bpad = jnp.concatenate([batch, jnp.full((NPAD - N,), 1 << 20, jnp.int32)])

    dstm, deg = _sc_deg(src, dst, zeros1)

    h1, h1p, dis = pl.pallas_call(
        _tc_pre_body,
        grid=(NRT,),
        in_specs=[_row_spec(F),
                  pl.BlockSpec((F, H), lambda i: (0, 0)),
                  _row_spec(1), _row_spec(1)],
        out_specs=[_row_spec(H), _row_spec(H), _row_spec(1)],
        out_shape=[jax.ShapeDtypeStruct((NPAD, H), f32),
                   jax.ShapeDtypeStruct((NPAD, H), f32),
                   jax.ShapeDtypeStruct((NPAD, 1), f32)],
    )(xpad, W1, deg[0][:, None], deg[1][:, None])

    def combine(agg, h, b, wn, has_next):
        outs = [jax.ShapeDtypeStruct((NPAD, H), f32)]
        out_specs = [_row_spec(H)]
        if has_next:
            outs += [jax.ShapeDtypeStruct((NPAD, H), f32),
                     jax.ShapeDtypeStruct((NPAD, H), f32)]
            out_specs += [_row_spec(H), _row_spec(H)]
        res = pl.pallas_call(
            functools.partial(_tc_combine_body, has_next),
            grid=(NRT,),
            in_specs=[_row_spec(H), _row_spec(H), _row_spec(H), _row_spec(1),
                      pl.BlockSpec((1, H), lambda i: (0, 0)),
                      pl.BlockSpec((H, H), lambda i: (0, 0))],
            out_specs=out_specs,
            out_shape=outs,
        )(agg[0], agg[1], h, dis, b[None, :], wn)
        return res if has_next else (res[0], None, None)

    agg1 = _sc_prop(h1p, src, dstm, zeros2)
    x1, h2, h2p = combine(agg1, h1, b1, W2, True)
    agg2 = _sc_prop(h2p, src, dstm, zeros2)
    x2, h3, h3p = combine(agg2, h2, b2, W3, True)
    agg3 = _sc_prop(h3p, src, dstm, zeros2)
    x3, _, _ = combine(agg3, h3, b3, W3, False)

    keys = x3[:, H - 1]
    k2d = keys.reshape(NRT, RT)
    b2d = bpad.reshape(NRT, RT)
    blo = bpad[::RT]
    bhi = bpad[RT - 1::RT]
    jlo = (jnp.searchsorted(bpad, blo, side="left") // RT).astype(jnp.int32)
    jhi = (-(-jnp.searchsorted(bpad, bhi, side="right") // RT)).astype(
        jnp.int32)
    rank = pl.pallas_call(
        _tc_rank_body,
        grid=(NRT,),
        in_specs=[_row_spec(1), _row_spec(1),
                  pl.BlockSpec((NRT, RT), lambda i: (0, 0)),
                  pl.BlockSpec((NRT, RT), lambda i: (0, 0)),
                  pl.BlockSpec(memory_space=pltpu.SMEM),
                  pl.BlockSpec(memory_space=pltpu.SMEM)],
        out_specs=_row_spec(1),
        out_shape=jax.ShapeDtypeStruct((NPAD, 1), jnp.int32),
    )(keys[:, None], bpad[:, None], k2d, b2d, jlo, jhi)

    pooled = _sc_pool(x1, x2, x3, bpad, rank[:, 0], zeros2)

    w1r = jnp.transpose(Wc1, (1, 3, 0, 2)).reshape(K, 96, 512)
    w2r = jnp.transpose(Wc2, (1, 3, 0, 2)).reshape(16, H, 160)
    p0 = pooled[0].reshape(K, G, 96)
    p1 = pooled[1].reshape(K, G, 96)
    m1 = jnp.kron(jnp.eye(16, dtype=f32), jnp.ones((H, 1), f32))
    m2 = jnp.kron(jnp.eye(10, dtype=f32), jnp.ones((16, 1), f32))
    mt1 = m1.T
    mt2 = m2.T

    pblk = pl.BlockSpec((1, G, 512), lambda i: (i, 0, 0))
    lblk = pl.BlockSpec((1, G, 16), lambda i: (i, 0, 0))
    sblk = pl.BlockSpec((G, 512), lambda i: (0, 0))
    mspec = pl.BlockSpec((512, 16), lambda i: (0, 0))
    mtspec = pl.BlockSpec((16, 512), lambda i: (0, 0))

    priors, ssum0 = pl.pallas_call(
        _tc_priors1_body,
        grid=(K,),
        in_specs=[pl.BlockSpec((1, G, 96), lambda i: (i, 0, 0)),
                  pl.BlockSpec((1, G, 96), lambda i: (i, 0, 0)),
                  pl.BlockSpec((1, 96, 512), lambda i: (i, 0, 0))],
        out_specs=[pblk, sblk],
        out_shape=[jax.ShapeDtypeStruct((K, G, 512), f32),
                   jax.ShapeDtypeStruct((G, 512), f32)],
    )(p0, p1, w1r)

    def squash_call(ssum, scale):
        return pl.pallas_call(
            functools.partial(_tc_squash_body, scale),
            out_shape=jax.ShapeDtypeStruct((G, 512), f32),
        )(ssum, m1, mt1)

    def route_step(first, lprev, out_prev):
        return pl.pallas_call(
            functools.partial(_tc_route_step_body, first),
            grid=(K,),
            in_specs=[pblk, lblk, sblk, mspec, mtspec],
            out_specs=[lblk, sblk],
            out_shape=[jax.ShapeDtypeStruct((K, G, 16), f32),
                       jax.ShapeDtypeStruct((G, 512), f32)],
        )(priors, lprev, out_prev, m1, mt1)

    out0 = squash_call(ssum0, 1.0 / 16.0)
    l1, ssum1 = route_step(True, jnp.zeros((K, G, 16), f32), out0)
    out1 = squash_call(ssum1, 1.0)
    _, ssum2 = route_step(False, l1, out1)
    out = pl.pallas_call(
        _tc_caps2_body,
        out_shape=jax.ShapeDtypeStruct((G, 10), f32),
    )(ssum2, w2r, m1, mt1, m2, mt2)
    return out
